# Initial kernel scaffold; baseline (speedup 1.0000x reference)
import jax
import jax.numpy as jnp
from jax.experimental import pallas as pl


def kernel(x, edge_index, W1l, b1l, W1r, b1r, W2l, b2l, W2r, b2r, Wlin, blin):
    raise NotImplementedError("write your pallas kernel here")


# speedup vs baseline: 7.7956x; 7.7956x over previous
import functools

import jax
import jax.numpy as jnp
from jax import lax
from jax.experimental import pallas as pl
from jax.experimental.pallas import tpu as pltpu
from jax.experimental.pallas import tpu_sc as plsc

N_NODES = 10000
N_EDGES = 320000
D_IN = 128
D_HID = 64
D_OUT = 2

NC = 2
NS = 16
NW = NC * NS
CHUNK = 128
CHUNKS_PER_TILE = -(-N_EDGES // (NW * CHUNK))
EDGES_PER_TILE = CHUNKS_PER_TILE * CHUNK
E_PAD = EDGES_PER_TILE * NW
N_PAD = 10112
ROWS_PER_TILE = N_PAD // NS
CW = 16
ROW_BLK = 1000


def _sc_scatter(with_cnt):
    mesh = plsc.VectorSubcoreMesh(core_axis_name="c", subcore_axis_name="s")
    agg_t = jax.ShapeDtypeStruct((NC, N_PAD, D_HID), jnp.float32)
    out_type = [agg_t] if with_cnt else agg_t
    scratch = [
        pltpu.VMEM((CHUNK,), jnp.int32),
        pltpu.VMEM((CHUNK,), jnp.int32),
        pltpu.VMEM((CHUNK,), jnp.int32),
        pltpu.VMEM((CHUNK,), jnp.int32),
        pltpu.VMEM((CHUNK, D_HID), jnp.float32),
        pltpu.VMEM((CHUNK, D_HID), jnp.float32),
        pltpu.VMEM_SHARED((N_PAD, D_HID), jnp.float32),
        pltpu.SemaphoreType.DMA,
        pltpu.SemaphoreType.DMA,
    ]
    if with_cnt:
        out_type.append(jax.ShapeDtypeStruct((NC, N_PAD, CW), jnp.float32))
        scratch += [
            pltpu.VMEM((CHUNK, CW), jnp.float32),
            pltpu.VMEM_SHARED((N_PAD, CW), jnp.float32),
        ]

    def body(*refs):
        if with_cnt:
            (p, src, dst, ones_h, z64, z16, agg_o, cnt_o,
             sA, sB, dA, dB, rA, rB, acc, semA, semB, ones_v, cacc) = refs
        else:
            (p, src, dst, z64, agg_o,
             sA, sB, dA, dB, rA, rB, acc, semA, semB) = refs

        cid = lax.axis_index("c")
        sid = lax.axis_index("s")
        wid = cid * NS + sid
        r0 = sid * ROWS_PER_TILE

        pltpu.sync_copy(z64, acc.at[pl.ds(r0, ROWS_PER_TILE)])
        if with_cnt:
            pltpu.sync_copy(z16, cacc.at[pl.ds(r0, ROWS_PER_TILE)])
            pltpu.sync_copy(ones_h, ones_v)
        plsc.subcore_barrier()

        base0 = wid * EDGES_PER_TILE

        def start(j, sv, rv, sem):
            pltpu.sync_copy(src.at[pl.ds(base0 + j * CHUNK, CHUNK)], sv)
            pltpu.async_copy(p.at[sv], rv, sem)

        def drain(sv, rv, sem):
            pltpu.make_async_copy(p.at[sv], rv, sem).wait()

        def scat(j, dv, rv):
            pltpu.sync_copy(dst.at[pl.ds(base0 + j * CHUNK, CHUNK)], dv)
            pltpu.sync_copy(rv, acc.at[dv], add=True)
            if with_cnt:
                pltpu.sync_copy(ones_v, cacc.at[dv], add=True)

        start(0, sA, rA, semA)

        @pl.loop(0, (CHUNKS_PER_TILE - 1) // 2)
        def _(it):
            jA = it * 2
            start(jA + 1, sB, rB, semB)
            drain(sA, rA, semA)
            scat(jA, dA, rA)
            start(jA + 2, sA, rA, semA)
            drain(sB, rB, semB)
            scat(jA + 1, dB, rB)

        drain(sA, rA, semA)
        scat(CHUNKS_PER_TILE - 1, dA, rA)

        plsc.subcore_barrier()
        pltpu.sync_copy(acc.at[pl.ds(r0, ROWS_PER_TILE)],
                        agg_o.at[cid].at[pl.ds(r0, ROWS_PER_TILE)])
        if with_cnt:
            pltpu.sync_copy(cacc.at[pl.ds(r0, ROWS_PER_TILE)],
                            cnt_o.at[cid].at[pl.ds(r0, ROWS_PER_TILE)])

    cp = pltpu.CompilerParams(use_tc_tiling_on_sc=False)
    return pl.kernel(body, out_type=out_type, mesh=mesh, scratch_types=scratch,
                     compiler_params=cp)


def _dense2(x, Wl, Wr, b2d):
    def tc_body(x_ref, wl_ref, wr_ref, b_ref, p_ref, q_ref):
        xb = x_ref[...]
        p_ref[...] = jnp.dot(xb, wl_ref[...],
                             preferred_element_type=jnp.float32,
                             precision=lax.Precision.HIGHEST)
        q_ref[...] = jnp.dot(xb, wr_ref[...],
                             preferred_element_type=jnp.float32,
                             precision=lax.Precision.HIGHEST) + b_ref[...]

    return pl.pallas_call(
        tc_body,
        grid=(N_NODES // ROW_BLK,),
        in_specs=[pl.BlockSpec((ROW_BLK, D_IN), lambda i: (i, 0)),
                  pl.BlockSpec((D_IN, D_HID), lambda i: (0, 0)),
                  pl.BlockSpec((D_IN, D_HID), lambda i: (0, 0)),
                  pl.BlockSpec((1, D_HID), lambda i: (0, 0))],
        out_specs=[pl.BlockSpec((ROW_BLK, D_HID), lambda i: (i, 0)),
                   pl.BlockSpec((ROW_BLK, D_HID), lambda i: (i, 0))],
        out_shape=[jax.ShapeDtypeStruct((N_NODES, D_HID), jnp.float32)] * 2,
    )(x, Wl, Wr, b2d)


def _mid(aggp, cntp, Q1, W2l, W2r, b2d):
    def tc_body(a_ref, c_ref, q_ref, wl_ref, wr_ref, b_ref, p_ref, q2_ref):
        a = a_ref[0] + a_ref[1]
        cnt = c_ref[0, :, 0:1] + c_ref[1, :, 0:1]
        inv = 1.0 / jnp.maximum(cnt, 1.0)
        h = jnp.maximum(a * inv + q_ref[...], 0.0)
        p_ref[...] = jnp.dot(h, wl_ref[...],
                             preferred_element_type=jnp.float32,
                             precision=lax.Precision.HIGHEST)
        q2_ref[...] = jnp.dot(h, wr_ref[...],
                              preferred_element_type=jnp.float32,
                              precision=lax.Precision.HIGHEST) + b_ref[...]

    return pl.pallas_call(
        tc_body,
        grid=(N_NODES // ROW_BLK,),
        in_specs=[pl.BlockSpec((NC, ROW_BLK, D_HID), lambda i: (0, i, 0)),
                  pl.BlockSpec((NC, ROW_BLK, CW), lambda i: (0, i, 0)),
                  pl.BlockSpec((ROW_BLK, D_HID), lambda i: (i, 0)),
                  pl.BlockSpec((D_HID, D_HID), lambda i: (0, 0)),
                  pl.BlockSpec((D_HID, D_HID), lambda i: (0, 0)),
                  pl.BlockSpec((1, D_HID), lambda i: (0, 0))],
        out_specs=[pl.BlockSpec((ROW_BLK, D_HID), lambda i: (i, 0)),
                   pl.BlockSpec((ROW_BLK, D_HID), lambda i: (i, 0))],
        out_shape=[jax.ShapeDtypeStruct((N_NODES, D_HID), jnp.float32)] * 2,
    )(aggp, cntp, Q1, W2l, W2r, b2d)


def _final(aggp, cntp, Q2, Wpad, bpad):
    def tc_body(a_ref, c_ref, q_ref, w_ref, b_ref, o_ref):
        a = a_ref[0] + a_ref[1]
        cnt = c_ref[0, :, 0:1] + c_ref[1, :, 0:1]
        inv = 1.0 / jnp.maximum(cnt, 1.0)
        h = jnp.maximum(a * inv + q_ref[...], 0.0)
        o_ref[...] = jnp.dot(h, w_ref[...],
                             preferred_element_type=jnp.float32,
                             precision=lax.Precision.HIGHEST) + b_ref[...]

    return pl.pallas_call(
        tc_body,
        grid=(N_NODES // ROW_BLK,),
        in_specs=[pl.BlockSpec((NC, ROW_BLK, D_HID), lambda i: (0, i, 0)),
                  pl.BlockSpec((NC, ROW_BLK, CW), lambda i: (0, i, 0)),
                  pl.BlockSpec((ROW_BLK, D_HID), lambda i: (i, 0)),
                  pl.BlockSpec((D_HID, 128), lambda i: (0, 0)),
                  pl.BlockSpec((1, 128), lambda i: (0, 0))],
        out_specs=pl.BlockSpec((ROW_BLK, 128), lambda i: (i, 0)),
        out_shape=jax.ShapeDtypeStruct((N_NODES, 128), jnp.float32),
    )(aggp, cntp, Q2, Wpad, bpad)


def kernel(x, edge_index, W1l, b1l, W1r, b1r, W2l, b2l, W2r, b2r, Wlin, blin):
    f32 = jnp.float32
    src = edge_index[0].astype(jnp.int32)
    dst = edge_index[1].astype(jnp.int32)
    npad = E_PAD - N_EDGES
    src_p = jnp.concatenate([src, jnp.zeros((npad,), jnp.int32)])
    dst_p = jnp.concatenate([dst, jnp.full((npad,), N_NODES, jnp.int32)])
    ones = jnp.ones((CHUNK, CW), f32)
    z64 = jnp.zeros((ROWS_PER_TILE, D_HID), f32)
    z16 = jnp.zeros((ROWS_PER_TILE, CW), f32)

    P1, Q1 = _dense2(x, W1l, W1r, (b1l + b1r).reshape(1, -1))
    agg1, cntp = _sc_scatter(True)(P1, src_p, dst_p, ones, z64, z16)
    P2, Q2 = _mid(agg1, cntp, Q1, W2l, W2r, (b2l + b2r).reshape(1, -1))
    agg2 = _sc_scatter(False)(P2, src_p, dst_p, z64)
    Wpad = jnp.pad(Wlin, ((0, 0), (0, 128 - D_OUT)))
    bpad = jnp.pad(blin, (0, 128 - D_OUT)).reshape(1, -1)
    outp = _final(agg2, cntp, Q2, Wpad, bpad)
    return outp[:, :D_OUT]
